# Initial kernel scaffold; baseline (speedup 1.0000x reference)
import jax
import jax.numpy as jnp
from jax.experimental import pallas as pl


def kernel(x, edge_index, edge_weight, W_self_0, W_neigh_0, b_0, W_self_1, W_neigh_1, b_1):
    raise NotImplementedError("write your pallas kernel here")


# speedup vs baseline: 4.0209x; 4.0209x over previous
import functools

import jax
import jax.numpy as jnp
from jax import lax
from jax.experimental import pallas as pl
from jax.experimental.pallas import tpu as pltpu
from jax.experimental.pallas import tpu_sc as plsc

DIM = 128
N_NODES = 10000
N_EDGES = 320000
NC = 2
NS = 16
NW = NC * NS
E_PER_TILE = N_EDGES // NW
CHUNK = 80
N_CHUNKS = E_PER_TILE // CHUNK
N_PAD = 10240
ROWS_PER_SUB = N_PAD // NS
DEG_W = 16
EXP_CHUNK = 80

_SC_MESH = plsc.VectorSubcoreMesh(core_axis_name="c", subcore_axis_name="s")


def _sc_agg_body(y_hbm, src_hbm, dst_hbm, w_hbm, z128_hbm,
                 part_hbm, degp_hbm,
                 accum_sh, deg_sh, src_v, dst_v, w_v, rows_v, ones_v,
                 deg1_v, sem):
    cid = lax.axis_index("c")
    sid = lax.axis_index("s")
    wid = sid * NC + cid
    base = wid * E_PER_TILE
    row0 = sid * ROWS_PER_SUB

    def ones_body(g, c):
        ones_v[pl.ds(g * 16, 16)] = jnp.full((16,), 1.0, jnp.float32)
        return c

    lax.fori_loop(0, CHUNK // 16, ones_body, 0)

    def zero_body(g, c):
        deg1_v[pl.ds(g * 16, 16)] = jnp.full((16,), 0.0, jnp.float32)
        return c

    lax.fori_loop(0, ROWS_PER_SUB // 16, zero_body, 0)

    pltpu.sync_copy(z128_hbm.at[pl.ds(row0, ROWS_PER_SUB)],
                    accum_sh.at[pl.ds(row0, ROWS_PER_SUB)])
    pltpu.sync_copy(deg1_v, deg_sh.at[pl.ds(row0, ROWS_PER_SUB)])
    plsc.subcore_barrier()

    def chunk_body(i, carry):
        off = base + i * CHUNK
        pltpu.sync_copy(src_hbm.at[pl.ds(off, CHUNK)], src_v)
        pltpu.sync_copy(dst_hbm.at[pl.ds(off, CHUNK)], dst_v)
        pltpu.sync_copy(w_hbm.at[pl.ds(off, CHUNK)], w_v)
        pltpu.async_copy(y_hbm.at[src_v], rows_v, sem).wait()

        def group_body(g, c2):
            wg = w_v[pl.ds(g * 16, 16)]
            for j in range(16):
                wv = wg[j]
                r = g * 16 + j
                for c in range(DIM // 16):
                    sl = pl.ds(c * 16, 16)
                    rows_v[r, sl] = rows_v[r, sl] * wv
            return c2

        lax.fori_loop(0, CHUNK // 16, group_body, 0)
        pltpu.sync_copy(rows_v, accum_sh.at[dst_v], add=True)
        pltpu.sync_copy(ones_v, deg_sh.at[dst_v], add=True)
        return carry

    lax.fori_loop(0, N_CHUNKS, chunk_body, 0)
    plsc.subcore_barrier()

    pltpu.sync_copy(accum_sh.at[pl.ds(row0, ROWS_PER_SUB)],
                    part_hbm.at[cid, pl.ds(row0, ROWS_PER_SUB)])

    pltpu.sync_copy(deg_sh.at[pl.ds(row0, ROWS_PER_SUB)], deg1_v)

    def expand_body(k, c2):
        for g in range(EXP_CHUNK // 16):
            dv = deg1_v[pl.ds(k * EXP_CHUNK + g * 16, 16)]
            for j in range(16):
                val = dv[j]
                r = g * 16 + j
                for c in range(DIM // 16):
                    sl = pl.ds(c * 16, 16)
                    rows_v[r, sl] = jnp.ones((16,), jnp.float32) * val
        pltpu.sync_copy(rows_v,
                        degp_hbm.at[cid, pl.ds(row0 + k * EXP_CHUNK, EXP_CHUNK)])
        return c2

    lax.fori_loop(0, ROWS_PER_SUB // EXP_CHUNK, expand_body, 0)


_sc_agg = pl.kernel(
    _sc_agg_body,
    out_type=(
        jax.ShapeDtypeStruct((NC, N_PAD, DIM), jnp.float32),
        jax.ShapeDtypeStruct((NC, N_PAD, DIM), jnp.float32),
    ),
    mesh=_SC_MESH,
    scratch_types=[
        pltpu.VMEM_SHARED((N_PAD, DIM), jnp.float32),
        pltpu.VMEM_SHARED((N_PAD,), jnp.float32),
        pltpu.VMEM((CHUNK,), jnp.int32),
        pltpu.VMEM((CHUNK,), jnp.int32),
        pltpu.VMEM((CHUNK,), jnp.float32),
        pltpu.VMEM((CHUNK, DIM), jnp.float32),
        pltpu.VMEM((CHUNK,), jnp.float32),
        pltpu.VMEM((ROWS_PER_SUB,), jnp.float32),
        pltpu.SemaphoreType.DMA,
    ],
)

ROW_BLK = 1000
GRID = N_NODES // ROW_BLK


def _tc_in_body(x_ref, ws_ref, wn_ref, b_ref, z_ref, y_ref):
    xb = x_ref[...]
    z_ref[...] = (
        jnp.dot(xb, ws_ref[...], preferred_element_type=jnp.float32) + b_ref[...]
    )
    y_ref[...] = jnp.dot(xb, wn_ref[...], preferred_element_type=jnp.float32)


def _tc_mid_body(z_ref, p_ref, dp_ref, ws_ref, wn_ref, b_ref, z1_ref, y1_ref):
    deg = dp_ref[0] + dp_ref[1]
    invd = 1.0 / jnp.maximum(deg, 1.0)
    agg = (p_ref[0] + p_ref[1]) * invd
    h = jnp.maximum(z_ref[...] + agg, 0.0)
    z1_ref[...] = (
        jnp.dot(h, ws_ref[...], preferred_element_type=jnp.float32) + b_ref[...]
    )
    y1_ref[...] = jnp.dot(h, wn_ref[...], preferred_element_type=jnp.float32)


def _tc_out_body(z_ref, p_ref, dp_ref, o_ref):
    deg = dp_ref[0] + dp_ref[1]
    invd = 1.0 / jnp.maximum(deg, 1.0)
    agg = (p_ref[0] + p_ref[1]) * invd
    o_ref[...] = jax.nn.sigmoid(z_ref[...] + agg)


_row_spec = pl.BlockSpec((ROW_BLK, DIM), lambda i: (i, 0))
_part_spec = pl.BlockSpec((NC, ROW_BLK, DIM), lambda i: (0, i, 0))
_degp_spec = pl.BlockSpec((NC, ROW_BLK, DIM), lambda i: (0, i, 0))
_w_spec = pl.BlockSpec((DIM, DIM), lambda i: (0, 0))
_b_spec = pl.BlockSpec((1, DIM), lambda i: (0, 0))

_tc_in = pl.pallas_call(
    _tc_in_body,
    grid=(GRID,),
    in_specs=[_row_spec, _w_spec, _w_spec, _b_spec],
    out_specs=[_row_spec, _row_spec],
    out_shape=[
        jax.ShapeDtypeStruct((N_NODES, DIM), jnp.float32),
        jax.ShapeDtypeStruct((N_NODES, DIM), jnp.float32),
    ],
)

_tc_mid = pl.pallas_call(
    _tc_mid_body,
    grid=(GRID,),
    in_specs=[_row_spec, _part_spec, _degp_spec, _w_spec, _w_spec, _b_spec],
    out_specs=[_row_spec, _row_spec],
    out_shape=[
        jax.ShapeDtypeStruct((N_NODES, DIM), jnp.float32),
        jax.ShapeDtypeStruct((N_NODES, DIM), jnp.float32),
    ],
)

_tc_out = pl.pallas_call(
    _tc_out_body,
    grid=(GRID,),
    in_specs=[_row_spec, _part_spec, _degp_spec],
    out_specs=_row_spec,
    out_shape=jax.ShapeDtypeStruct((N_NODES, DIM), jnp.float32),
)


@jax.jit
def kernel(x, edge_index, edge_weight, W_self_0, W_neigh_0, b_0,
           W_self_1, W_neigh_1, b_1):
    src = edge_index[0].astype(jnp.int32)
    dst = edge_index[1].astype(jnp.int32)
    w = edge_weight.astype(jnp.float32)
    z128 = jnp.zeros((N_PAD, DIM), jnp.float32)

    z0, y0 = _tc_in(x, W_self_0, W_neigh_0, b_0.reshape(1, DIM))
    part0, degp = _sc_agg(y0, src, dst, w, z128)
    z1, y1 = _tc_mid(z0, part0, degp, W_self_1, W_neigh_1, b_1.reshape(1, DIM))
    part1, _ = _sc_agg(y1, src, dst, w, z128)
    return _tc_out(z1, part1, degp)


# speedup vs baseline: 10.8166x; 2.6901x over previous
import jax
import jax.numpy as jnp
from jax import lax
from jax.experimental import pallas as pl
from jax.experimental.pallas import tpu as pltpu
from jax.experimental.pallas import tpu_sc as plsc

DIM = 128
N_NODES = 10000
N_EDGES = 320000
NC = 2
NS = 16
NW = NC * NS
E_PER_TILE = N_EDGES // NW
CHUNK = 80
N_CHUNKS = E_PER_TILE // CHUNK
N_PAD = 10240
ROWS_PER_SUB = N_PAD // NS
EXP_CHUNK = 80
NROW = 3
NIDX = 6
MAIN_CHUNKS = 120

_SC_MESH = plsc.VectorSubcoreMesh(core_axis_name="c", subcore_axis_name="s")


def _multiply(rows_ref, w_ref):
    def group_body(g, c2):
        wg = w_ref[pl.ds(g * 16, 16)]
        for j in range(16):
            wv = wg[j]
            r = g * 16 + j
            for c in range(DIM // 16):
                sl = pl.ds(c * 16, 16)
                rows_ref[r, sl] = rows_ref[r, sl] * wv
        return c2

    lax.fori_loop(0, CHUNK // 16, group_body, 0)


def _make_sc_body(with_deg):
    def wrapped(y_hbm, src_hbm, dst_hbm, w_hbm, z128_hbm, part_hbm, degp_hbm,
                accum_sh, deg_sh,
                r0, r1, r2, s0, s1, s2, s3, s4, s5,
                d0, d1, d2, d3, d4, d5, w0, w1, w2, w3, w4, w5,
                ones_v, deg1_v,
                g0, g1, g2, c0, c1, c2, e0, e1, e2,
                i0, i1, i2, i3, i4, i5):
        rows = (r0, r1, r2)
        srcb = (s0, s1, s2, s3, s4, s5)
        dstb = (d0, d1, d2, d3, d4, d5)
        wb = (w0, w1, w2, w3, w4, w5)
        gsem = (g0, g1, g2)
        ssem = (c0, c1, c2)
        dsem = (e0, e1, e2)
        isem = (i0, i1, i2, i3, i4, i5)

        cid = lax.axis_index("c")
        sid = lax.axis_index("s")
        wid = sid * NC + cid
        base = wid * E_PER_TILE
        row0 = sid * ROWS_PER_SUB

        def issue_idx(c_dyn, b):
            off = base + c_dyn * CHUNK
            pltpu.async_copy(src_hbm.at[pl.ds(off, CHUNK)], srcb[b], isem[b])
            pltpu.async_copy(dst_hbm.at[pl.ds(off, CHUNK)], dstb[b], isem[b])
            pltpu.async_copy(w_hbm.at[pl.ds(off, CHUNK)], wb[b], isem[b])

        def wait_idx(b):
            pltpu.make_async_copy(src_hbm.at[pl.ds(0, CHUNK)], srcb[b], isem[b]).wait()
            pltpu.make_async_copy(dst_hbm.at[pl.ds(0, CHUNK)], dstb[b], isem[b]).wait()
            pltpu.make_async_copy(w_hbm.at[pl.ds(0, CHUNK)], wb[b], isem[b]).wait()

        def issue_gather(br, bi):
            pltpu.async_copy(y_hbm.at[srcb[bi]], rows[br], gsem[br])

        def wait_gather(br, bi):
            pltpu.make_async_copy(y_hbm.at[srcb[bi]], rows[br], gsem[br]).wait()

        def issue_scatter(br, bi):
            pltpu.async_copy(rows[br], accum_sh.at[dstb[bi]], ssem[br], add=True)
            if with_deg:
                pltpu.async_copy(ones_v, deg_sh.at[dstb[bi]], dsem[br], add=True)

        def wait_scatter(br, bi):
            pltpu.make_async_copy(rows[br], accum_sh.at[dstb[bi]], ssem[br]).wait()
            if with_deg:
                pltpu.make_async_copy(ones_v, deg_sh.at[dstb[bi]], dsem[br]).wait()

        if with_deg:
            def ones_body(g, c):
                ones_v[pl.ds(g * 16, 16)] = jnp.full((16,), 1.0, jnp.float32)
                return c

            lax.fori_loop(0, CHUNK // 16, ones_body, 0)

            def zero_body(g, c):
                deg1_v[pl.ds(g * 16, 16)] = jnp.full((16,), 0.0, jnp.float32)
                return c

            lax.fori_loop(0, EXP_CHUNK // 16, zero_body, 0)

            def deg_zero_body(k, c):
                pltpu.sync_copy(deg1_v,
                                deg_sh.at[pl.ds(row0 + k * EXP_CHUNK, EXP_CHUNK)])
                return c

            lax.fori_loop(0, ROWS_PER_SUB // EXP_CHUNK, deg_zero_body, 0)

        pltpu.sync_copy(z128_hbm.at[pl.ds(row0, ROWS_PER_SUB)],
                        accum_sh.at[pl.ds(row0, ROWS_PER_SUB)])
        plsc.subcore_barrier()

        for t in range(4):
            issue_idx(jnp.int32(t), t)
        wait_idx(0)
        wait_idx(1)
        issue_gather(0, 0)
        issue_gather(1, 1)

        def slot(c_dyn, s, do_ga, do_ix, guard_first):
            br, bi = s % NROW, s % NIDX
            wait_gather(br, bi)
            _multiply(rows[br], wb[bi])
            if guard_first:
                @pl.when(c_dyn >= 1)
                def _():
                    wait_scatter((s + 2) % NROW, (s + 5) % NIDX)
            else:
                wait_scatter((s + 2) % NROW, (s + 5) % NIDX)
            if do_ga:
                wait_idx((s + 2) % NIDX)
                issue_gather((s + 2) % NROW, (s + 2) % NIDX)
            issue_scatter(br, bi)
            if do_ix:
                issue_idx(c_dyn + 4, (s + 4) % NIDX)

        def main_loop(k, carry):
            for s in range(NIDX):
                slot(k * NIDX + s, s, True, True, s == 0)
            return carry

        lax.fori_loop(0, MAIN_CHUNKS // NIDX, main_loop, 0)

        slot(jnp.int32(120), 0, True, True, False)
        slot(jnp.int32(121), 1, True, False, False)
        slot(jnp.int32(122), 2, True, False, False)
        slot(jnp.int32(123), 3, False, False, False)
        slot(jnp.int32(124), 4, False, False, False)
        wait_scatter(124 % NROW, 124 % NIDX)
        plsc.subcore_barrier()

        pltpu.sync_copy(accum_sh.at[pl.ds(row0, ROWS_PER_SUB)],
                        part_hbm.at[cid, pl.ds(row0, ROWS_PER_SUB)])

        if with_deg:
            def expand_body(k, c2):
                roff = row0 + k * EXP_CHUNK
                pltpu.sync_copy(deg_sh.at[pl.ds(roff, EXP_CHUNK)], deg1_v)
                for g in range(EXP_CHUNK // 16):
                    dv = deg1_v[pl.ds(g * 16, 16)]
                    for j in range(16):
                        val = dv[j]
                        r = g * 16 + j
                        for c in range(DIM // 16):
                            sl = pl.ds(c * 16, 16)
                            rows[0][r, sl] = jnp.ones((16,), jnp.float32) * val
                pltpu.sync_copy(rows[0], degp_hbm.at[cid, pl.ds(roff, EXP_CHUNK)])
                return c2

            lax.fori_loop(0, ROWS_PER_SUB // EXP_CHUNK, expand_body, 0)

    return wrapped


def _make_sc_kernel(with_deg):
    scratch = [
        pltpu.VMEM_SHARED((N_PAD, DIM), jnp.float32),
        pltpu.VMEM_SHARED((N_PAD,), jnp.float32),
    ]
    scratch += [pltpu.VMEM((CHUNK, DIM), jnp.float32)] * NROW
    scratch += [pltpu.VMEM((CHUNK,), jnp.int32)] * NIDX
    scratch += [pltpu.VMEM((CHUNK,), jnp.int32)] * NIDX
    scratch += [pltpu.VMEM((CHUNK,), jnp.float32)] * NIDX
    scratch += [
        pltpu.VMEM((CHUNK,), jnp.float32),
        pltpu.VMEM((EXP_CHUNK,), jnp.float32),
    ]
    scratch += [pltpu.SemaphoreType.DMA] * (NROW * 3 + NIDX)
    return pl.kernel(
        _make_sc_body(with_deg),
        out_type=(
            jax.ShapeDtypeStruct((NC, N_PAD, DIM), jnp.float32),
            jax.ShapeDtypeStruct((NC, N_PAD, DIM), jnp.float32),
        ),
        mesh=_SC_MESH,
        scratch_types=scratch,
    )


_sc_agg_deg = _make_sc_kernel(True)
_sc_agg = _make_sc_kernel(False)

ROW_BLK = 1000
GRID = N_NODES // ROW_BLK


def _tc_in_body(x_ref, ws_ref, wn_ref, b_ref, z_ref, y_ref):
    xb = x_ref[...]
    z_ref[...] = (
        jnp.dot(xb, ws_ref[...], preferred_element_type=jnp.float32) + b_ref[...]
    )
    y_ref[...] = jnp.dot(xb, wn_ref[...], preferred_element_type=jnp.float32)


def _tc_mid_body(z_ref, p_ref, dp_ref, ws_ref, wn_ref, b_ref, z1_ref, y1_ref):
    deg = dp_ref[0] + dp_ref[1]
    invd = 1.0 / jnp.maximum(deg, 1.0)
    agg = (p_ref[0] + p_ref[1]) * invd
    h = jnp.maximum(z_ref[...] + agg, 0.0)
    z1_ref[...] = (
        jnp.dot(h, ws_ref[...], preferred_element_type=jnp.float32) + b_ref[...]
    )
    y1_ref[...] = jnp.dot(h, wn_ref[...], preferred_element_type=jnp.float32)


def _tc_out_body(z_ref, p_ref, dp_ref, o_ref):
    deg = dp_ref[0] + dp_ref[1]
    invd = 1.0 / jnp.maximum(deg, 1.0)
    agg = (p_ref[0] + p_ref[1]) * invd
    o_ref[...] = jax.nn.sigmoid(z_ref[...] + agg)


_row_spec = pl.BlockSpec((ROW_BLK, DIM), lambda i: (i, 0))
_part_spec = pl.BlockSpec((NC, ROW_BLK, DIM), lambda i: (0, i, 0))
_degp_spec = pl.BlockSpec((NC, ROW_BLK, DIM), lambda i: (0, i, 0))
_w_spec = pl.BlockSpec((DIM, DIM), lambda i: (0, 0))
_b_spec = pl.BlockSpec((1, DIM), lambda i: (0, 0))

_tc_in = pl.pallas_call(
    _tc_in_body,
    grid=(GRID,),
    in_specs=[_row_spec, _w_spec, _w_spec, _b_spec],
    out_specs=[_row_spec, _row_spec],
    out_shape=[
        jax.ShapeDtypeStruct((N_NODES, DIM), jnp.float32),
        jax.ShapeDtypeStruct((N_NODES, DIM), jnp.float32),
    ],
)

_tc_mid = pl.pallas_call(
    _tc_mid_body,
    grid=(GRID,),
    in_specs=[_row_spec, _part_spec, _degp_spec, _w_spec, _w_spec, _b_spec],
    out_specs=[_row_spec, _row_spec],
    out_shape=[
        jax.ShapeDtypeStruct((N_NODES, DIM), jnp.float32),
        jax.ShapeDtypeStruct((N_NODES, DIM), jnp.float32),
    ],
)

_tc_out = pl.pallas_call(
    _tc_out_body,
    grid=(GRID,),
    in_specs=[_row_spec, _part_spec, _degp_spec],
    out_specs=_row_spec,
    out_shape=jax.ShapeDtypeStruct((N_NODES, DIM), jnp.float32),
)


@jax.jit
def kernel(x, edge_index, edge_weight, W_self_0, W_neigh_0, b_0,
           W_self_1, W_neigh_1, b_1):
    src = edge_index[0].astype(jnp.int32)
    dst = edge_index[1].astype(jnp.int32)
    w = edge_weight.astype(jnp.float32)
    z128 = jnp.zeros((N_PAD, DIM), jnp.float32)

    z0, y0 = _tc_in(x, W_self_0, W_neigh_0, b_0.reshape(1, DIM))
    part0, degp = _sc_agg_deg(y0, src, dst, w, z128)
    z1, y1 = _tc_mid(z0, part0, degp, W_self_1, W_neigh_1, b_1.reshape(1, DIM))
    part1, _ = _sc_agg(y1, src, dst, w, z128)
    return _tc_out(z1, part1, degp)


# speedup vs baseline: 11.1842x; 1.0340x over previous
import jax
import jax.numpy as jnp
from jax import lax
from jax.experimental import pallas as pl
from jax.experimental.pallas import tpu as pltpu
from jax.experimental.pallas import tpu_sc as plsc

DIM = 128
N_NODES = 10000
N_EDGES = 320000
NC = 2
NS = 16
NW = NC * NS
E_PER_TILE = N_EDGES // NW
CHUNK = 80
N_CHUNKS = E_PER_TILE // CHUNK
N_PAD = 10240
ROWS_PER_SUB = N_PAD // NS
EXP_CHUNK = 80
NROW = 3
NIDX = 6
MAIN_CHUNKS = 120

_SC_MESH = plsc.VectorSubcoreMesh(core_axis_name="c", subcore_axis_name="s")


def _multiply(rows_ref, w_ref, unrolled):
    def group_body(g, c2):
        wg = w_ref[pl.ds(g * 16, 16)]
        for j in range(16):
            wv = wg[j]
            r = g * 16 + j
            for c in range(DIM // 16):
                sl = pl.ds(c * 16, 16)
                rows_ref[r, sl] = rows_ref[r, sl] * wv
        return c2

    if unrolled:
        for g in range(CHUNK // 16):
            group_body(g, 0)
    else:
        lax.fori_loop(0, CHUNK // 16, group_body, 0)


def _make_sc_body(with_deg):
    def wrapped(y_hbm, src_hbm, dst_hbm, w_hbm, z128_hbm, part_hbm, degp_hbm,
                accum_sh, deg_sh,
                r0, r1, r2, s0, s1, s2, s3, s4, s5,
                d0, d1, d2, d3, d4, d5, w0, w1, w2, w3, w4, w5,
                ones_v, deg1_v,
                g0, g1, g2, c0, c1, c2, e0, e1, e2,
                i0, i1, i2, i3, i4, i5):
        rows = (r0, r1, r2)
        srcb = (s0, s1, s2, s3, s4, s5)
        dstb = (d0, d1, d2, d3, d4, d5)
        wb = (w0, w1, w2, w3, w4, w5)
        gsem = (g0, g1, g2)
        ssem = (c0, c1, c2)
        dsem = (e0, e1, e2)
        isem = (i0, i1, i2, i3, i4, i5)

        cid = lax.axis_index("c")
        sid = lax.axis_index("s")
        wid = sid * NC + cid
        base = wid * E_PER_TILE
        row0 = sid * ROWS_PER_SUB

        def issue_idx(c_dyn, b):
            off = base + c_dyn * CHUNK
            pltpu.async_copy(src_hbm.at[pl.ds(off, CHUNK)], srcb[b], isem[b])
            pltpu.async_copy(dst_hbm.at[pl.ds(off, CHUNK)], dstb[b], isem[b])
            pltpu.async_copy(w_hbm.at[pl.ds(off, CHUNK)], wb[b], isem[b])

        def wait_idx(b):
            pltpu.make_async_copy(src_hbm.at[pl.ds(0, CHUNK)], srcb[b], isem[b]).wait()
            pltpu.make_async_copy(dst_hbm.at[pl.ds(0, CHUNK)], dstb[b], isem[b]).wait()
            pltpu.make_async_copy(w_hbm.at[pl.ds(0, CHUNK)], wb[b], isem[b]).wait()

        def issue_gather(br, bi):
            pltpu.async_copy(y_hbm.at[srcb[bi]], rows[br], gsem[br])

        def wait_gather(br, bi):
            pltpu.make_async_copy(y_hbm.at[srcb[bi]], rows[br], gsem[br]).wait()

        def issue_scatter(br, bi):
            pltpu.async_copy(rows[br], accum_sh.at[dstb[bi]], ssem[br], add=True)
            if with_deg:
                pltpu.async_copy(ones_v, deg_sh.at[dstb[bi]], dsem[br], add=True)

        def wait_scatter(br, bi):
            pltpu.make_async_copy(rows[br], accum_sh.at[dstb[bi]], ssem[br]).wait()
            if with_deg:
                pltpu.make_async_copy(ones_v, deg_sh.at[dstb[bi]], dsem[br]).wait()

        if with_deg:
            def ones_body(g, c):
                ones_v[pl.ds(g * 16, 16)] = jnp.full((16,), 1.0, jnp.float32)
                return c

            lax.fori_loop(0, CHUNK // 16, ones_body, 0)

            def zero_body(g, c):
                deg1_v[pl.ds(g * 16, 16)] = jnp.full((16,), 0.0, jnp.float32)
                return c

            lax.fori_loop(0, EXP_CHUNK // 16, zero_body, 0)

            def deg_zero_body(k, c):
                pltpu.sync_copy(deg1_v,
                                deg_sh.at[pl.ds(row0 + k * EXP_CHUNK, EXP_CHUNK)])
                return c

            lax.fori_loop(0, ROWS_PER_SUB // EXP_CHUNK, deg_zero_body, 0)

        pltpu.sync_copy(z128_hbm.at[pl.ds(row0, ROWS_PER_SUB)],
                        accum_sh.at[pl.ds(row0, ROWS_PER_SUB)])
        plsc.subcore_barrier()

        for t in range(4):
            issue_idx(jnp.int32(t), t)
        wait_idx(0)
        wait_idx(1)
        issue_gather(0, 0)
        issue_gather(1, 1)

        def slot(c_dyn, s, do_ga, do_ix, guard_first):
            br, bi = s % NROW, s % NIDX
            wait_gather(br, bi)
            _multiply(rows[br], wb[bi], False)
            if guard_first:
                @pl.when(c_dyn >= 1)
                def _():
                    wait_scatter((s + 2) % NROW, (s + 5) % NIDX)
            else:
                wait_scatter((s + 2) % NROW, (s + 5) % NIDX)
            if do_ga:
                wait_idx((s + 2) % NIDX)
                issue_gather((s + 2) % NROW, (s + 2) % NIDX)
            issue_scatter(br, bi)
            if do_ix:
                issue_idx(c_dyn + 4, (s + 4) % NIDX)

        def main_loop(k, carry):
            for s in range(NIDX):
                slot(k * NIDX + s, s, True, True, s == 0)
            return carry

        lax.fori_loop(0, MAIN_CHUNKS // NIDX, main_loop, 0)

        slot(jnp.int32(120), 0, True, True, False)
        slot(jnp.int32(121), 1, True, False, False)
        slot(jnp.int32(122), 2, True, False, False)
        slot(jnp.int32(123), 3, False, False, False)
        slot(jnp.int32(124), 4, False, False, False)
        wait_scatter(124 % NROW, 124 % NIDX)
        plsc.subcore_barrier()

        pltpu.sync_copy(accum_sh.at[pl.ds(row0, ROWS_PER_SUB)],
                        part_hbm.at[cid, pl.ds(row0, ROWS_PER_SUB)])

        if with_deg:
            def expand_body(k, c2):
                roff = row0 + k * EXP_CHUNK
                pltpu.sync_copy(deg_sh.at[pl.ds(roff, EXP_CHUNK)], deg1_v)
                for g in range(EXP_CHUNK // 16):
                    dv = deg1_v[pl.ds(g * 16, 16)]
                    for j in range(16):
                        val = dv[j]
                        r = g * 16 + j
                        for c in range(DIM // 16):
                            sl = pl.ds(c * 16, 16)
                            rows[0][r, sl] = jnp.ones((16,), jnp.float32) * val
                pltpu.sync_copy(rows[0], degp_hbm.at[cid, pl.ds(roff, EXP_CHUNK)])
                return c2

            lax.fori_loop(0, ROWS_PER_SUB // EXP_CHUNK, expand_body, 0)

    return wrapped


def _make_sc_kernel(with_deg):
    scratch = [
        pltpu.VMEM_SHARED((N_PAD, DIM), jnp.float32),
        pltpu.VMEM_SHARED((N_PAD,), jnp.float32),
    ]
    scratch += [pltpu.VMEM((CHUNK, DIM), jnp.float32)] * NROW
    scratch += [pltpu.VMEM((CHUNK,), jnp.int32)] * NIDX
    scratch += [pltpu.VMEM((CHUNK,), jnp.int32)] * NIDX
    scratch += [pltpu.VMEM((CHUNK,), jnp.float32)] * NIDX
    scratch += [
        pltpu.VMEM((CHUNK,), jnp.float32),
        pltpu.VMEM((EXP_CHUNK,), jnp.float32),
    ]
    scratch += [pltpu.SemaphoreType.DMA] * (NROW * 3 + NIDX)
    return pl.kernel(
        _make_sc_body(with_deg),
        out_type=(
            jax.ShapeDtypeStruct((NC, N_PAD, DIM), jnp.float32),
            jax.ShapeDtypeStruct((NC, N_PAD, DIM), jnp.float32),
        ),
        mesh=_SC_MESH,
        scratch_types=scratch,
    )


_sc_agg_deg = _make_sc_kernel(True)
_sc_agg = _make_sc_kernel(False)

ROW_BLK = 1000
GRID = N_NODES // ROW_BLK


def _tc_in_body(x_ref, ws_ref, b_ref, z_ref):
    z_ref[...] = (
        jnp.dot(x_ref[...], ws_ref[...], preferred_element_type=jnp.float32)
        + b_ref[...]
    )


def _tc_mid_body(z_ref, p_ref, dp_ref, wn0_ref, ws_ref, wn_ref, b_ref,
                 z1_ref, y1_ref):
    deg = dp_ref[0] + dp_ref[1]
    invd = 1.0 / jnp.maximum(deg, 1.0)
    aggx = (p_ref[0] + p_ref[1]) * invd
    agg = jnp.dot(aggx, wn0_ref[...], preferred_element_type=jnp.float32)
    h = jnp.maximum(z_ref[...] + agg, 0.0)
    z1_ref[...] = (
        jnp.dot(h, ws_ref[...], preferred_element_type=jnp.float32) + b_ref[...]
    )
    y1_ref[...] = jnp.dot(h, wn_ref[...], preferred_element_type=jnp.float32)


def _tc_out_body(z_ref, p_ref, dp_ref, o_ref):
    deg = dp_ref[0] + dp_ref[1]
    invd = 1.0 / jnp.maximum(deg, 1.0)
    agg = (p_ref[0] + p_ref[1]) * invd
    o_ref[...] = jax.nn.sigmoid(z_ref[...] + agg)


_row_spec = pl.BlockSpec((ROW_BLK, DIM), lambda i: (i, 0))
_part_spec = pl.BlockSpec((NC, ROW_BLK, DIM), lambda i: (0, i, 0))
_degp_spec = pl.BlockSpec((NC, ROW_BLK, DIM), lambda i: (0, i, 0))
_w_spec = pl.BlockSpec((DIM, DIM), lambda i: (0, 0))
_b_spec = pl.BlockSpec((1, DIM), lambda i: (0, 0))

_tc_in = pl.pallas_call(
    _tc_in_body,
    grid=(GRID,),
    in_specs=[_row_spec, _w_spec, _b_spec],
    out_specs=_row_spec,
    out_shape=jax.ShapeDtypeStruct((N_NODES, DIM), jnp.float32),
)

_tc_mid = pl.pallas_call(
    _tc_mid_body,
    grid=(GRID,),
    in_specs=[_row_spec, _part_spec, _degp_spec, _w_spec, _w_spec, _w_spec, _b_spec],
    out_specs=[_row_spec, _row_spec],
    out_shape=[
        jax.ShapeDtypeStruct((N_NODES, DIM), jnp.float32),
        jax.ShapeDtypeStruct((N_NODES, DIM), jnp.float32),
    ],
)

_tc_out = pl.pallas_call(
    _tc_out_body,
    grid=(GRID,),
    in_specs=[_row_spec, _part_spec, _degp_spec],
    out_specs=_row_spec,
    out_shape=jax.ShapeDtypeStruct((N_NODES, DIM), jnp.float32),
)


@jax.jit
def kernel(x, edge_index, edge_weight, W_self_0, W_neigh_0, b_0,
           W_self_1, W_neigh_1, b_1):
    src = edge_index[0].astype(jnp.int32)
    dst = edge_index[1].astype(jnp.int32)
    w = edge_weight.astype(jnp.float32)
    z128 = jnp.zeros((N_PAD, DIM), jnp.float32)

    z0 = _tc_in(x, W_self_0, b_0.reshape(1, DIM))
    part0, degp = _sc_agg_deg(x, src, dst, w, z128)
    z1, y1 = _tc_mid(z0, part0, degp, W_neigh_0, W_self_1, W_neigh_1,
                     b_1.reshape(1, DIM))
    part1, _ = _sc_agg(y1, src, dst, w, z128)
    return _tc_out(z1, part1, degp)
